# baseline (speedup 1.0000x reference)
import functools

import jax
import jax.numpy as jnp
from jax import lax
from jax.experimental import pallas as pl
from jax.experimental.pallas import tpu as pltpu
from jax.experimental.pallas import tpu_sc as plsc

E = 64
TOP_K = 8
N_GROUP = 8
GROUP_SIZE = E // N_GROUP
TOPK_GROUP = 4
D_MODEL = 1024
D_FF = 512
ROUTED_SCALING = 2.5

T = 128
L = 16
NW_USED = T // L

NEG_INF = float("-inf")


def _scores_kernel(hidden_ref, gw_ref, bias_ref, scores_ref, s4c_ref):
    logits_t = jax.lax.dot_general(
        gw_ref[...], hidden_ref[...], (((1,), (1,)), ((), ())),
        preferred_element_type=jnp.float32)
    scores_t = jax.nn.sigmoid(logits_t)
    scores_ref[...] = scores_t
    s4c_ref[...] = scores_t + bias_ref[...]


def _sc_routing_body(scores_hbm, s4c_hbm, combine_hbm,
                     s_v, s4c_v, ms_v, sel_v, out_v, sem):
    cid = lax.axis_index("c")
    sid = lax.axis_index("s")
    wid = sid * 2 + cid

    @pl.when(wid < NW_USED)
    def _work():
        base = wid * L
        pltpu.async_copy(scores_hbm, s_v, sem).wait()
        pltpu.async_copy(s4c_hbm, s4c_v, sem).wait()

        minf = jnp.full((L,), NEG_INF, jnp.float32)

        group_scores = []
        for g in range(N_GROUP):
            m1 = s4c_v[g * GROUP_SIZE, pl.ds(base, L)]
            m2 = minf
            for k in range(1, GROUP_SIZE):
                v = s4c_v[g * GROUP_SIZE + k, pl.ds(base, L)]
                m2 = jnp.maximum(m2, jnp.minimum(m1, v))
                m1 = jnp.maximum(m1, v)
            group_scores.append(m1 + m2)

        one = jnp.ones((L,), jnp.float32)
        zero = jnp.zeros((L,), jnp.float32)

        gsel = [zero for _ in range(N_GROUP)]
        for _ in range(TOPK_GROUP):
            m = group_scores[0]
            for g in range(1, N_GROUP):
                m = jnp.maximum(m, group_scores[g])
            found = zero
            for g in range(N_GROUP):
                eq = jnp.where(group_scores[g] == m, one, zero)
                is_first = eq * (one - found)
                found = found + is_first
                gsel[g] = gsel[g] + is_first
                group_scores[g] = jnp.where(is_first > 0.5, minf,
                                            group_scores[g])

        for j in range(E):
            ms_v[j, :] = jnp.where(gsel[j // GROUP_SIZE] > 0.5,
                                   s4c_v[j, pl.ds(base, L)], minf)
            sel_v[j, :] = zero

        for _ in range(TOP_K):
            def max_body(j, m):
                return jnp.maximum(m, ms_v[j, :])
            m = lax.fori_loop(0, E, max_body, minf)

            def mark_body(j, found):
                v = ms_v[j, :]
                eq = jnp.where(v == m, one, zero)
                is_first = eq * (one - found)
                ms_v[j, :] = jnp.where(is_first > 0.5, minf, v)
                sel_v[j, :] = sel_v[j, :] + is_first
                return found + is_first
            lax.fori_loop(0, E, mark_body, zero)

        def wsum_body(j, acc):
            return acc + s_v[j, pl.ds(base, L)] * sel_v[j, :]
        wsum = lax.fori_loop(0, E, wsum_body, jnp.zeros((L,), jnp.float32))
        scale = ROUTED_SCALING / (wsum + 1e-20)

        for j in range(E):
            out_v[j, :] = s_v[j, pl.ds(base, L)] * sel_v[j, :] * scale

        pltpu.async_copy(out_v, combine_hbm.at[wid], sem).wait()


NBUF = 4


def _moe_kernel(hidden_ref, combine_ref, wgu_hbm, wd_hbm, out_ref,
                wgu_buf, wd_buf, wgu_sem, wd_sem):
    def start(e):
        b = e % NBUF
        pltpu.make_async_copy(wgu_hbm.at[e], wgu_buf.at[b], wgu_sem.at[b]).start()
        pltpu.make_async_copy(wd_hbm.at[e], wd_buf.at[b], wd_sem.at[b]).start()

    for e in range(NBUF):
        start(e)

    combine = combine_ref[...]
    hidden = hidden_ref[...]
    lane = jax.lax.broadcasted_iota(jnp.int32, (T, E), 1)
    acc = jnp.zeros_like(out_ref)
    for e in range(E):
        b = e % NBUF
        pltpu.make_async_copy(wgu_hbm.at[e], wgu_buf.at[b], wgu_sem.at[b]).wait()
        pltpu.make_async_copy(wd_hbm.at[e], wd_buf.at[b], wd_sem.at[b]).wait()
        gu = jnp.dot(hidden, wgu_buf[b], preferred_element_type=jnp.float32)
        gate = gu[:, :D_FF]
        up = gu[:, D_FF:]
        h = jax.nn.silu(gate) * up
        cw = jnp.sum(jnp.where(lane == e, combine, 0.0), axis=1,
                     keepdims=True)
        acc += jnp.dot(h * cw, wd_buf[b], preferred_element_type=jnp.float32)
        if e + NBUF < E:
            start(e + NBUF)
    out_ref[...] = acc


@jax.jit
def kernel(hidden_states, gate_weight, e_score_correction_bias, w_gate_up, w_down):
    bias_col = e_score_correction_bias.reshape(E, 1)

    scores_t, s4c_t = pl.pallas_call(
        _scores_kernel,
        out_shape=(
            jax.ShapeDtypeStruct((E, T), jnp.float32),
            jax.ShapeDtypeStruct((E, T), jnp.float32),
        ),
    )(hidden_states, gate_weight, bias_col)

    sc_routing = functools.partial(
        pl.kernel,
        mesh=plsc.VectorSubcoreMesh(core_axis_name="c", subcore_axis_name="s"),
        out_type=jax.ShapeDtypeStruct((NW_USED, E, L), jnp.float32),
        scratch_types=[
            pltpu.VMEM((E, T), jnp.float32),
            pltpu.VMEM((E, T), jnp.float32),
            pltpu.VMEM((E, L), jnp.float32),
            pltpu.VMEM((E, L), jnp.float32),
            pltpu.VMEM((E, L), jnp.float32),
            pltpu.SemaphoreType.DMA,
        ],
    )(_sc_routing_body)
    combine_wel = sc_routing(scores_t, s4c_t)
    combine = combine_wel.transpose(0, 2, 1).reshape(T, E)

    out = pl.pallas_call(
        _moe_kernel,
        in_specs=[
            pl.BlockSpec(memory_space=pltpu.MemorySpace.VMEM),
            pl.BlockSpec(memory_space=pltpu.MemorySpace.VMEM),
            pl.BlockSpec(memory_space=pl.ANY),
            pl.BlockSpec(memory_space=pl.ANY),
        ],
        out_specs=pl.BlockSpec(memory_space=pltpu.MemorySpace.VMEM),
        out_shape=jax.ShapeDtypeStruct((T, D_MODEL), jnp.float32),
        scratch_shapes=[
            pltpu.VMEM((NBUF, D_MODEL, 2 * D_FF), jnp.float32),
            pltpu.VMEM((NBUF, D_FF, D_MODEL), jnp.float32),
            pltpu.SemaphoreType.DMA((NBUF,)),
            pltpu.SemaphoreType.DMA((NBUF,)),
        ],
    )(hidden_states, combine, w_gate_up, w_down)
    return out


# speedup vs baseline: 1.2673x; 1.2673x over previous
import jax
import jax.numpy as jnp
from jax.experimental import pallas as pl
from jax.experimental.pallas import tpu as pltpu

E = 64
TOP_K = 8
N_GROUP = 8
GROUP_SIZE = E // N_GROUP
TOPK_GROUP = 4
D_MODEL = 1024
D_FF = 512
ROUTED_SCALING = 2.5


def _routing(hidden, gw, bias):
    logits = jax.lax.dot_general(
        hidden, gw, (((1,), (1,)), ((), ())),
        preferred_element_type=jnp.float32)
    scores = jax.nn.sigmoid(logits)
    s4c = scores + bias

    group_cols = []
    for g in range(N_GROUP):
        m1 = s4c[:, g * GROUP_SIZE:g * GROUP_SIZE + 1]
        m2 = jnp.full_like(m1, -jnp.inf)
        for k in range(1, GROUP_SIZE):
            v = s4c[:, g * GROUP_SIZE + k:g * GROUP_SIZE + k + 1]
            m2 = jnp.maximum(m2, jnp.minimum(m1, v))
            m1 = jnp.maximum(m1, v)
        group_cols.append(m1 + m2)
    group_scores = jnp.concatenate(group_cols, axis=1)

    iota_r8 = jax.lax.broadcasted_iota(jnp.int32, (N_GROUP, N_GROUP), 0)
    iota_c8 = jax.lax.broadcasted_iota(jnp.int32, (N_GROUP, N_GROUP), 1)
    cumtri8 = (iota_r8 <= iota_c8).astype(jnp.float32)
    work = group_scores
    gmask = jnp.zeros_like(group_scores)
    for _ in range(TOPK_GROUP):
        m = jnp.max(work, axis=1, keepdims=True)
        ism = (work == m).astype(jnp.float32)
        cs = jax.lax.dot(ism, cumtri8, preferred_element_type=jnp.float32)
        first = jnp.where((ism > 0) & (cs == 1.0), 1.0, 0.0)
        gmask = gmask + first
        work = jnp.where(first > 0, -jnp.inf, work)

    iota_g = jax.lax.broadcasted_iota(jnp.int32, (N_GROUP, E), 0)
    iota_e = jax.lax.broadcasted_iota(jnp.int32, (N_GROUP, E), 1)
    expand = (iota_e // GROUP_SIZE == iota_g).astype(jnp.float32)
    score_mask = jax.lax.dot(gmask, expand, preferred_element_type=jnp.float32)
    masked = jnp.where(score_mask > 0, s4c, -jnp.inf)

    iota_rE = jax.lax.broadcasted_iota(jnp.int32, (E, E), 0)
    iota_cE = jax.lax.broadcasted_iota(jnp.int32, (E, E), 1)
    cumtriE = (iota_rE <= iota_cE).astype(jnp.float32)
    sel = jnp.zeros_like(masked)
    work = masked
    for _ in range(TOP_K):
        m = jnp.max(work, axis=1, keepdims=True)
        ism = (work == m).astype(jnp.float32)
        cs = jax.lax.dot(ism, cumtriE, preferred_element_type=jnp.float32)
        first = jnp.where((ism > 0) & (cs == 1.0), 1.0, 0.0)
        sel = sel + first
        work = jnp.where(first > 0, -jnp.inf, work)

    w = scores * sel
    wsum = jnp.sum(w, axis=1, keepdims=True) + 1e-20
    return (ROUTED_SCALING / wsum) * w


NBUF = 4


def _moe_kernel(hidden_ref, gw_ref, bias_ref, wgu_hbm, wd_hbm, out_ref,
                wgu_buf, wd_buf, wgu_sem, wd_sem):
    def start(e):
        b = e % NBUF
        pltpu.make_async_copy(wgu_hbm.at[e], wgu_buf.at[b], wgu_sem.at[b]).start()
        pltpu.make_async_copy(wd_hbm.at[e], wd_buf.at[b], wd_sem.at[b]).start()

    for e in range(NBUF):
        start(e)

    combine = _routing(hidden_ref[...], gw_ref[...], bias_ref[...])

    hidden = hidden_ref[...]
    lane = jax.lax.broadcasted_iota(jnp.int32, (combine.shape[0], E), 1)
    acc = jnp.zeros_like(out_ref)
    for e0 in range(0, E, 2):
        for e in (e0, e0 + 1):
            b = e % NBUF
            pltpu.make_async_copy(wgu_hbm.at[e], wgu_buf.at[b], wgu_sem.at[b]).wait()
            pltpu.make_async_copy(wd_hbm.at[e], wd_buf.at[b], wd_sem.at[b]).wait()
        for e in (e0, e0 + 1):
            b = e % NBUF
            gu = jnp.dot(hidden, wgu_buf[b], preferred_element_type=jnp.float32)
            gate = gu[:, :D_FF]
            up = gu[:, D_FF:]
            h = jax.nn.silu(gate) * up
            cw = jnp.sum(jnp.where(lane == e, combine, 0.0), axis=1,
                         keepdims=True)
            acc += jnp.dot(h * cw, wd_buf[b], preferred_element_type=jnp.float32)
            if e + NBUF < E:
                start(e + NBUF)
    out_ref[...] = acc


@jax.jit
def kernel(hidden_states, gate_weight, e_score_correction_bias, w_gate_up, w_down):
    T = hidden_states.shape[0]
    bias2d = e_score_correction_bias.reshape(1, E)

    out = pl.pallas_call(
        _moe_kernel,
        in_specs=[
            pl.BlockSpec(memory_space=pltpu.MemorySpace.VMEM),
            pl.BlockSpec(memory_space=pltpu.MemorySpace.VMEM),
            pl.BlockSpec(memory_space=pltpu.MemorySpace.VMEM),
            pl.BlockSpec(memory_space=pl.ANY),
            pl.BlockSpec(memory_space=pl.ANY),
        ],
        out_specs=pl.BlockSpec(memory_space=pltpu.MemorySpace.VMEM),
        out_shape=jax.ShapeDtypeStruct((T, D_MODEL), jnp.float32),
        scratch_shapes=[
            pltpu.VMEM((NBUF, D_MODEL, 2 * D_FF), jnp.float32),
            pltpu.VMEM((NBUF, D_FF, D_MODEL), jnp.float32),
            pltpu.SemaphoreType.DMA((NBUF,)),
            pltpu.SemaphoreType.DMA((NBUF,)),
        ],
    )(hidden_states, gate_weight, bias2d, w_gate_up, w_down)
    return out
